# baseline (speedup 1.0000x reference)
import functools
import jax
import jax.numpy as jnp
from jax import lax
from jax.experimental import pallas as pl
from jax.experimental.pallas import tpu as pltpu
from jax.experimental.pallas import tpu_sc as plsc

N = 10000
E = 320000
B = 16
L = 256
DF = 128
DO = 128
DD = 80

NP = 10240
HR = 80
NC = 2
NS = 16
CH = 128
NCHUNK = E // CH
FULL = NCHUNK // NS
REM = NCHUNK - FULL * NS

_mesh = plsc.VectorSubcoreMesh(core_axis_name="c", subcore_axis_name="s")


def _leaky(x):
    return jnp.where(x >= 0, x, 0.01 * x)


@functools.partial(
    pl.kernel,
    out_type=(
        jax.ShapeDtypeStruct((NP,), jnp.float32),
        jax.ShapeDtypeStruct((NP,), jnp.float32),
    ),
    mesh=_mesh,
    scratch_types=dict(
        shared=pltpu.VMEM_SHARED((NS, NP), jnp.float32),
        hist=pltpu.VMEM((NP,), jnp.float32),
        idxbuf=pltpu.VMEM((E // NS,), jnp.int32),
        colbuf=pltpu.VMEM((NS, NP // NS), jnp.float32),
        sumbuf=pltpu.VMEM((NP // NS,), jnp.float32),
    ),
    compiler_params=pltpu.CompilerParams(needs_layout_passes=False),
)
def _deg_kernel(dst1, dst2, zeros1d, cnt1, cnt2,
                shared, hist, idxbuf, colbuf, sumbuf):
    cid = lax.axis_index("c")
    sid = lax.axis_index("s")
    per_tile = E // NS
    seg = NP // NS

    pltpu.sync_copy(zeros1d, hist)

    def run(dst_hbm, out_hbm):
        pltpu.sync_copy(dst_hbm.at[pl.ds(sid * per_tile, per_tile)], idxbuf)
        ones = jnp.full((16,), 1.0, jnp.float32)

        def body(i, _):
            idx = idxbuf[pl.ds(i * 16, 16)]
            plsc.addupdate_scatter(hist, [idx], ones)
            return 0

        lax.fori_loop(0, per_tile // 16, body, 0)
        pltpu.sync_copy(hist, shared.at[sid])
        plsc.subcore_barrier()
        for t in range(NS):
            pltpu.sync_copy(shared.at[t, pl.ds(sid * seg, seg)],
                            colbuf.at[t])

        def red(j, _):
            s = jnp.zeros((16,), jnp.float32)
            for t in range(NS):
                s = s + colbuf[t, pl.ds(j * 16, 16)]
            sumbuf[pl.ds(j * 16, 16)] = s
            return 0

        lax.fori_loop(0, seg // 16, red, 0)
        pltpu.sync_copy(sumbuf, out_hbm.at[pl.ds(sid * seg, seg)])

    @pl.when(cid == 0)
    def _():
        run(dst1, cnt1)

    @pl.when(cid == 1)
    def _():
        run(dst2, cnt2)


@functools.partial(
    pl.kernel,
    out_type=(
        jax.ShapeDtypeStruct((NP, DF), jnp.float32),
        jax.ShapeDtypeStruct((NP, DF), jnp.float32),
    ),
    mesh=_mesh,
    scratch_types=dict(
        acc_sh=pltpu.VMEM_SHARED((NP, DF), jnp.float32),
        sdbuf=pltpu.VMEM((2, CH), jnp.int32),
        rows=pltpu.VMEM((CH, DF), jnp.float32),
        sem=pltpu.SemaphoreType.DMA,
    ),
    compiler_params=pltpu.CompilerParams(needs_layout_passes=False),
)
def _agg_kernel(g1, g2, sd1, sd2, zrows, acc1, acc2,
                acc_sh, sdbuf, rows, sem):
    cid = lax.axis_index("c")
    sid = lax.axis_index("s")
    zr = NP // NS
    pltpu.sync_copy(zrows, acc_sh.at[pl.ds(sid * zr, zr)])
    plsc.subcore_barrier()

    def run(g_hbm, sd_hbm, out_hbm):
        def body(i, _):
            c = i * NS + sid

            @pl.when(c < NCHUNK)
            def _():
                pltpu.sync_copy(sd_hbm.at[c], sdbuf)
                pltpu.async_copy(g_hbm.at[sdbuf.at[0]], rows, sem).wait()
                pltpu.sync_copy(rows, acc_sh.at[sdbuf.at[1]], add=True)

            return 0

        lax.fori_loop(0, FULL + 1, body, 0)
        plsc.subcore_barrier()
        orows = NP // NS
        pltpu.sync_copy(
            acc_sh.at[pl.ds(sid * orows, orows)],
            out_hbm.at[pl.ds(sid * orows, orows)],
        )

    @pl.when(cid == 0)
    def _():
        run(g1, sd1, acc1)

    @pl.when(cid == 1)
    def _():
        run(g2, sd2, acc2)


def _hg_body(x1, w1, c1, x2, w2, c2, h1, g1, h2, g2):
    for x, w, c, h, g in ((x1, w1, c1, h1, g1), (x2, w2, c2, h2, g2)):
        hv = jnp.dot(x[...], w[...], preferred_element_type=jnp.float32)
        dis = lax.rsqrt(c[...] + 1.0)
        h[...] = hv
        g[...] = hv * dis


def _hg(x1, W1, cnt1, x2, W2, cnt2):
    nb = 10
    rb = N // nb
    spec_x = pl.BlockSpec((rb, DF), lambda i: (i, 0))
    spec_w = pl.BlockSpec((DF, DF), lambda i: (0, 0))
    spec_c = pl.BlockSpec((rb, 1), lambda i: (i, 0))
    return pl.pallas_call(
        _hg_body,
        grid=(nb,),
        in_specs=[spec_x, spec_w, spec_c, spec_x, spec_w, spec_c],
        out_specs=[spec_x, spec_x, spec_x, spec_x],
        out_shape=[jax.ShapeDtypeStruct((N, DF), jnp.float32)] * 4,
    )(x1, W1, cnt1, x2, W2, cnt2)


def _mas_body(d_ref, w_ref, b_ref, len_ref, o_ref):
    y = jnp.dot(d_ref[0, 0], w_ref[0], preferred_element_type=jnp.float32)
    y = _leaky(y + b_ref[0])
    pos = lax.broadcasted_iota(jnp.int32, (L, 1), 0)
    y = jnp.where(pos < len_ref[0, 0], y, -1e30)
    o_ref[0, 0] = jnp.max(y, axis=0, keepdims=True)


def _mas(data, wts, bias, lens):
    return pl.pallas_call(
        _mas_body,
        grid=(4, B),
        in_specs=[
            pl.BlockSpec((1, 1, L, DD), lambda b, g: (b, g, 0, 0)),
            pl.BlockSpec((1, DD, DO), lambda b, g: (b, 0, 0)),
            pl.BlockSpec((1, 1, DO), lambda b, g: (b, 0, 0)),
            pl.BlockSpec((1, 1, L, 1), lambda b, g: (b, g, 0, 0)),
        ],
        out_specs=pl.BlockSpec((1, 1, 1, DO), lambda b, g: (b, g, 0, 0)),
        out_shape=jax.ShapeDtypeStruct((4, B, 1, DO), jnp.float32),
    )(data, wts, bias, lens)


def _combine_body(acc1, h1, c1, bt1, bg1, wf1, bf1,
                  acc2, h2, c2, bt2, bg2, wf2, bf2,
                  mas, wfin, bfin, out):
    feats = []
    for acc, h, c, bt, bg, wf, bf in (
        (acc1, h1, c1, bt1, bg1, wf1, bf1),
        (acc2, h2, c2, bt2, bg2, wf2, bf2),
    ):
        deg = c[...] + 1.0
        dis = lax.rsqrt(deg)
        xn = _leaky(dis * acc[...] + h[...] / deg + bg[...])
        gid = lax.broadcasted_iota(jnp.int32, (B, N), 0)
        m = (gid == bt[...]).astype(jnp.float32)
        sums = jnp.dot(m, xn, preferred_element_type=jnp.float32)
        cnts = jnp.sum(m, axis=1, keepdims=True)
        mean = sums / jnp.maximum(cnts, 1.0)
        feats.append(_leaky(jnp.dot(mean, wf[...],
                                    preferred_element_type=jnp.float32)
                            + bf[...]))
    for i in range(4):
        feats.append(mas[i, :, 0, :])
    comb = jnp.concatenate(feats, axis=1)
    out[...] = jnp.dot(comb, wfin[...],
                       preferred_element_type=jnp.float32) + bfin[...]


def _combine(acc1, h1, cnt1, bt1, bg1, wf1, bf1,
             acc2, h2, cnt2, bt2, bg2, wf2, bf2,
             mas, wfin, bfin):
    args = (acc1, h1, cnt1, bt1, bg1, wf1, bf1,
            acc2, h2, cnt2, bt2, bg2, wf2, bf2,
            mas, wfin, bfin)
    return pl.pallas_call(
        _combine_body,
        out_shape=jax.ShapeDtypeStruct((B, 1), jnp.float32),
    )(*args)


def kernel(pro1_x, pro1_edge_index, pro1_batch, pro2_x, pro2_edge_index,
           pro2_batch, mas1_straight, mas1_flipped, mas2_straight,
           mas2_flipped, mas1_straight_lengths, mas1_flipped_lengths,
           mas2_straight_lengths, mas2_flipped_lengths,
           W_gcn1, b_gcn1, W_gcn2, b_gcn2, W_fc1, b_fc1, W_fc2, b_fc2,
           W_m1s, b_m1s, W_m1f, b_m1f, W_m2s, b_m2s, W_m2f, b_m2f,
           W_final, b_final):
    src1, dst1 = pro1_edge_index[0], pro1_edge_index[1]
    src2, dst2 = pro2_edge_index[0], pro2_edge_index[1]

    zeros1d = jnp.zeros((NP,), jnp.float32)
    cnt1_p, cnt2_p = _deg_kernel(dst1, dst2, zeros1d)
    cnt1 = cnt1_p[:N].reshape(N, 1)
    cnt2 = cnt2_p[:N].reshape(N, 1)

    h1, g1, h2, g2 = _hg(pro1_x, W_gcn1, cnt1, pro2_x, W_gcn2, cnt2)

    sd1 = jnp.stack([src1.reshape(NCHUNK, CH), dst1.reshape(NCHUNK, CH)], 1)
    sd2 = jnp.stack([src2.reshape(NCHUNK, CH), dst2.reshape(NCHUNK, CH)], 1)

    zrows = jnp.zeros((NP // NS, DF), jnp.float32)
    acc1, acc2 = _agg_kernel(g1, g2, sd1, sd2, zrows)
    acc1, acc2 = acc1[:N], acc2[:N]

    mas_data = jnp.stack([mas1_straight, mas1_flipped,
                          mas2_straight, mas2_flipped])
    mas_w = jnp.stack([W_m1s.T, W_m1f.T, W_m2s.T, W_m2f.T])
    mas_b = jnp.stack([b_m1s, b_m1f, b_m2s, b_m2f]).reshape(4, 1, DO)
    mas_len = jnp.stack([mas1_straight_lengths, mas1_flipped_lengths,
                         mas2_straight_lengths, mas2_flipped_lengths])
    mas_len = jnp.broadcast_to(mas_len[:, :, None, None], (4, B, L, 1))
    mas_out = _mas(mas_data, mas_w, mas_b, mas_len)

    return _combine(
        acc1, h1, cnt1, pro1_batch.reshape(1, N), b_gcn1.reshape(1, DF),
        W_fc1.T, b_fc1.reshape(1, DO),
        acc2, h2, cnt2, pro2_batch.reshape(1, N), b_gcn2.reshape(1, DF),
        W_fc2.T, b_fc2.reshape(1, DO),
        mas_out, W_final.T, b_final.reshape(1, 1))


# speedup vs baseline: 1.1610x; 1.1610x over previous
import functools
import jax
import jax.numpy as jnp
from jax import lax
from jax.experimental import pallas as pl
from jax.experimental.pallas import tpu as pltpu
from jax.experimental.pallas import tpu_sc as plsc

N = 10000
E = 320000
B = 16
L = 256
DF = 128
DO = 128
DD = 80

NP = 10240
HR = 80
NC = 2
NS = 16
CH = 128
NCHUNK = E // CH
FULL = NCHUNK // NS
REM = NCHUNK - FULL * NS

_mesh = plsc.VectorSubcoreMesh(core_axis_name="c", subcore_axis_name="s")


def _leaky(x):
    return jnp.where(x >= 0, x, 0.01 * x)


@functools.partial(
    pl.kernel,
    out_type=(
        jax.ShapeDtypeStruct((NP,), jnp.float32),
        jax.ShapeDtypeStruct((NP,), jnp.float32),
    ),
    mesh=_mesh,
    scratch_types=dict(
        shared=pltpu.VMEM_SHARED((NS, NP), jnp.float32),
        hist=pltpu.VMEM((NP,), jnp.float32),
        idxbuf=pltpu.VMEM((E // NS,), jnp.int32),
        colbuf=pltpu.VMEM((NS, NP // NS), jnp.float32),
        sumbuf=pltpu.VMEM((NP // NS,), jnp.float32),
    ),
    compiler_params=pltpu.CompilerParams(needs_layout_passes=False),
)
def _deg_kernel(dst1, dst2, zeros1d, cnt1, cnt2,
                shared, hist, idxbuf, colbuf, sumbuf):
    cid = lax.axis_index("c")
    sid = lax.axis_index("s")
    per_tile = E // NS
    seg = NP // NS

    pltpu.sync_copy(zeros1d, hist)

    def run(dst_hbm, out_hbm):
        pltpu.sync_copy(dst_hbm.at[pl.ds(sid * per_tile, per_tile)], idxbuf)
        ones = jnp.full((16,), 1.0, jnp.float32)

        def body(i, _):
            idx = idxbuf[pl.ds(i * 16, 16)]
            plsc.addupdate_scatter(hist, [idx], ones)
            return 0

        lax.fori_loop(0, per_tile // 16, body, 0)
        pltpu.sync_copy(hist, shared.at[sid])
        plsc.subcore_barrier()
        for t in range(NS):
            pltpu.sync_copy(shared.at[t, pl.ds(sid * seg, seg)],
                            colbuf.at[t])

        def red(j, _):
            s = jnp.zeros((16,), jnp.float32)
            for t in range(NS):
                s = s + colbuf[t, pl.ds(j * 16, 16)]
            sumbuf[pl.ds(j * 16, 16)] = s
            return 0

        lax.fori_loop(0, seg // 16, red, 0)
        pltpu.sync_copy(sumbuf, out_hbm.at[pl.ds(sid * seg, seg)])

    @pl.when(cid == 0)
    def _():
        run(dst1, cnt1)

    @pl.when(cid == 1)
    def _():
        run(dst2, cnt2)


@functools.partial(
    pl.kernel,
    out_type=(
        jax.ShapeDtypeStruct((NP, DF), jnp.float32),
        jax.ShapeDtypeStruct((NP, DF), jnp.float32),
    ),
    mesh=_mesh,
    scratch_types=dict(
        acc_sh=pltpu.VMEM_SHARED((NP, DF), jnp.float32),
        sd0=pltpu.VMEM((2, CH), jnp.int32),
        sd1b=pltpu.VMEM((2, CH), jnp.int32),
        rows=pltpu.VMEM((CH, DF), jnp.float32),
        sem=pltpu.SemaphoreType.DMA,
        ssem0=pltpu.SemaphoreType.DMA,
        ssem1=pltpu.SemaphoreType.DMA,
    ),
    compiler_params=pltpu.CompilerParams(needs_layout_passes=False),
)
def _agg_kernel(g1, g2, sd1, sd2, zrows, acc1, acc2,
                acc_sh, sd0, sd1b, rows, sem, ssem0, ssem1):
    cid = lax.axis_index("c")
    sid = lax.axis_index("s")
    zr = NP // NS
    pltpu.sync_copy(zrows, acc_sh.at[pl.ds(sid * zr, zr)])
    plsc.subcore_barrier()

    def run(g_hbm, sd_hbm, out_hbm):
        pltpu.async_copy(sd_hbm.at[sid], sd0, ssem0)

        def body(j, _):
            c0 = (2 * j) * NS + sid
            c1 = (2 * j + 1) * NS + sid

            @pl.when(c0 < NCHUNK)
            def _():
                pltpu.make_async_copy(sd_hbm.at[c0], sd0, ssem0).wait()

                @pl.when(c1 < NCHUNK)
                def _():
                    pltpu.async_copy(sd_hbm.at[c1], sd1b, ssem1)

                pltpu.async_copy(g_hbm.at[sd0.at[0]], rows, sem).wait()
                pltpu.sync_copy(rows, acc_sh.at[sd0.at[1]], add=True)

            @pl.when(c1 < NCHUNK)
            def _():
                pltpu.make_async_copy(sd_hbm.at[c1], sd1b, ssem1).wait()

                @pl.when(c0 + 2 * NS < NCHUNK)
                def _():
                    pltpu.async_copy(sd_hbm.at[c0 + 2 * NS], sd0, ssem0)

                pltpu.async_copy(g_hbm.at[sd1b.at[0]], rows, sem).wait()
                pltpu.sync_copy(rows, acc_sh.at[sd1b.at[1]], add=True)

            return 0

        lax.fori_loop(0, (FULL + 2) // 2, body, 0)
        plsc.subcore_barrier()
        orows = NP // NS
        pltpu.sync_copy(
            acc_sh.at[pl.ds(sid * orows, orows)],
            out_hbm.at[pl.ds(sid * orows, orows)],
        )

    @pl.when(cid == 0)
    def _():
        run(g1, sd1, acc1)

    @pl.when(cid == 1)
    def _():
        run(g2, sd2, acc2)


def _hg_body(x1, w1, c1, x2, w2, c2, h1, g1, h2, g2):
    for x, w, c, h, g in ((x1, w1, c1, h1, g1), (x2, w2, c2, h2, g2)):
        hv = jnp.dot(x[...], w[...], preferred_element_type=jnp.float32)
        dis = lax.rsqrt(c[...] + 1.0)
        h[...] = hv
        g[...] = hv * dis


def _hg(x1, W1, cnt1, x2, W2, cnt2):
    nb = 10
    rb = N // nb
    spec_x = pl.BlockSpec((rb, DF), lambda i: (i, 0))
    spec_w = pl.BlockSpec((DF, DF), lambda i: (0, 0))
    spec_c = pl.BlockSpec((rb, 1), lambda i: (i, 0))
    return pl.pallas_call(
        _hg_body,
        grid=(nb,),
        in_specs=[spec_x, spec_w, spec_c, spec_x, spec_w, spec_c],
        out_specs=[spec_x, spec_x, spec_x, spec_x],
        out_shape=[jax.ShapeDtypeStruct((N, DF), jnp.float32)] * 4,
    )(x1, W1, cnt1, x2, W2, cnt2)


def _mas_body(d_ref, w_ref, b_ref, len_ref, o_ref):
    y = jnp.dot(d_ref[0, 0], w_ref[0], preferred_element_type=jnp.float32)
    y = _leaky(y + b_ref[0])
    pos = lax.broadcasted_iota(jnp.int32, (L, 1), 0)
    y = jnp.where(pos < len_ref[0, 0], y, -1e30)
    o_ref[0, 0] = jnp.max(y, axis=0, keepdims=True)


def _mas(data, wts, bias, lens):
    return pl.pallas_call(
        _mas_body,
        grid=(4, B),
        in_specs=[
            pl.BlockSpec((1, 1, L, DD), lambda b, g: (b, g, 0, 0)),
            pl.BlockSpec((1, DD, DO), lambda b, g: (b, 0, 0)),
            pl.BlockSpec((1, 1, DO), lambda b, g: (b, 0, 0)),
            pl.BlockSpec((1, 1, L, 1), lambda b, g: (b, g, 0, 0)),
        ],
        out_specs=pl.BlockSpec((1, 1, 1, DO), lambda b, g: (b, g, 0, 0)),
        out_shape=jax.ShapeDtypeStruct((4, B, 1, DO), jnp.float32),
    )(data, wts, bias, lens)


def _combine_body(acc1, h1, c1, bt1, bg1, wf1, bf1,
                  acc2, h2, c2, bt2, bg2, wf2, bf2,
                  mas, wfin, bfin, out):
    feats = []
    for acc, h, c, bt, bg, wf, bf in (
        (acc1, h1, c1, bt1, bg1, wf1, bf1),
        (acc2, h2, c2, bt2, bg2, wf2, bf2),
    ):
        deg = c[...] + 1.0
        dis = lax.rsqrt(deg)
        xn = _leaky(dis * acc[...] + h[...] / deg + bg[...])
        gid = lax.broadcasted_iota(jnp.int32, (B, N), 0)
        m = (gid == bt[...]).astype(jnp.float32)
        sums = jnp.dot(m, xn, preferred_element_type=jnp.float32)
        cnts = jnp.sum(m, axis=1, keepdims=True)
        mean = sums / jnp.maximum(cnts, 1.0)
        feats.append(_leaky(jnp.dot(mean, wf[...],
                                    preferred_element_type=jnp.float32)
                            + bf[...]))
    for i in range(4):
        feats.append(mas[i, :, 0, :])
    comb = jnp.concatenate(feats, axis=1)
    out[...] = jnp.dot(comb, wfin[...],
                       preferred_element_type=jnp.float32) + bfin[...]


def _combine(acc1, h1, cnt1, bt1, bg1, wf1, bf1,
             acc2, h2, cnt2, bt2, bg2, wf2, bf2,
             mas, wfin, bfin):
    args = (acc1, h1, cnt1, bt1, bg1, wf1, bf1,
            acc2, h2, cnt2, bt2, bg2, wf2, bf2,
            mas, wfin, bfin)
    return pl.pallas_call(
        _combine_body,
        out_shape=jax.ShapeDtypeStruct((B, 1), jnp.float32),
    )(*args)


def kernel(pro1_x, pro1_edge_index, pro1_batch, pro2_x, pro2_edge_index,
           pro2_batch, mas1_straight, mas1_flipped, mas2_straight,
           mas2_flipped, mas1_straight_lengths, mas1_flipped_lengths,
           mas2_straight_lengths, mas2_flipped_lengths,
           W_gcn1, b_gcn1, W_gcn2, b_gcn2, W_fc1, b_fc1, W_fc2, b_fc2,
           W_m1s, b_m1s, W_m1f, b_m1f, W_m2s, b_m2s, W_m2f, b_m2f,
           W_final, b_final):
    src1, dst1 = pro1_edge_index[0], pro1_edge_index[1]
    src2, dst2 = pro2_edge_index[0], pro2_edge_index[1]

    zeros1d = jnp.zeros((NP,), jnp.float32)
    cnt1_p, cnt2_p = _deg_kernel(dst1, dst2, zeros1d)
    cnt1 = cnt1_p[:N].reshape(N, 1)
    cnt2 = cnt2_p[:N].reshape(N, 1)

    h1, g1, h2, g2 = _hg(pro1_x, W_gcn1, cnt1, pro2_x, W_gcn2, cnt2)

    sd1 = jnp.stack([src1.reshape(NCHUNK, CH), dst1.reshape(NCHUNK, CH)], 1)
    sd2 = jnp.stack([src2.reshape(NCHUNK, CH), dst2.reshape(NCHUNK, CH)], 1)

    zrows = jnp.zeros((NP // NS, DF), jnp.float32)
    acc1, acc2 = _agg_kernel(g1, g2, sd1, sd2, zrows)
    acc1, acc2 = acc1[:N], acc2[:N]

    mas_data = jnp.stack([mas1_straight, mas1_flipped,
                          mas2_straight, mas2_flipped])
    mas_w = jnp.stack([W_m1s.T, W_m1f.T, W_m2s.T, W_m2f.T])
    mas_b = jnp.stack([b_m1s, b_m1f, b_m2s, b_m2f]).reshape(4, 1, DO)
    mas_len = jnp.stack([mas1_straight_lengths, mas1_flipped_lengths,
                         mas2_straight_lengths, mas2_flipped_lengths])
    mas_len = jnp.broadcast_to(mas_len[:, :, None, None], (4, B, L, 1))
    mas_out = _mas(mas_data, mas_w, mas_b, mas_len)

    return _combine(
        acc1, h1, cnt1, pro1_batch.reshape(1, N), b_gcn1.reshape(1, DF),
        W_fc1.T, b_fc1.reshape(1, DO),
        acc2, h2, cnt2, pro2_batch.reshape(1, N), b_gcn2.reshape(1, DF),
        W_fc2.T, b_fc2.reshape(1, DO),
        mas_out, W_final.T, b_final.reshape(1, 1))


# speedup vs baseline: 1.4523x; 1.2510x over previous
import functools
import jax
import jax.numpy as jnp
from jax import lax
from jax.experimental import pallas as pl
from jax.experimental.pallas import tpu as pltpu
from jax.experimental.pallas import tpu_sc as plsc

N = 10000
E = 320000
B = 16
L = 256
DF = 128
DO = 128
DD = 80

NP = 10240
HR = 80
NC = 2
NS = 16
CH = 128
NCHUNK = E // CH
FULL = NCHUNK // NS
REM = NCHUNK - FULL * NS

_mesh = plsc.VectorSubcoreMesh(core_axis_name="c", subcore_axis_name="s")


def _leaky(x):
    return jnp.where(x >= 0, x, 0.01 * x)


@functools.partial(
    pl.kernel,
    out_type=(
        jax.ShapeDtypeStruct((NP,), jnp.float32),
        jax.ShapeDtypeStruct((NP,), jnp.float32),
    ),
    mesh=_mesh,
    scratch_types=dict(
        shared=pltpu.VMEM_SHARED((NS, NP), jnp.float32),
        hist=pltpu.VMEM((NP,), jnp.float32),
        idxbuf=pltpu.VMEM((E // NS,), jnp.int32),
        colbuf=pltpu.VMEM((NS, NP // NS), jnp.float32),
        sumbuf=pltpu.VMEM((NP // NS,), jnp.float32),
    ),
    compiler_params=pltpu.CompilerParams(needs_layout_passes=False),
)
def _deg_kernel(dst1, dst2, zeros1d, cnt1, cnt2,
                shared, hist, idxbuf, colbuf, sumbuf):
    cid = lax.axis_index("c")
    sid = lax.axis_index("s")
    per_tile = E // NS
    seg = NP // NS

    pltpu.sync_copy(zeros1d, hist)

    def run(dst_hbm, out_hbm):
        pltpu.sync_copy(dst_hbm.at[pl.ds(sid * per_tile, per_tile)], idxbuf)
        ones = jnp.full((16,), 1.0, jnp.float32)

        def body(i, _):
            idx = idxbuf[pl.ds(i * 16, 16)]
            plsc.addupdate_scatter(hist, [idx], ones)
            return 0

        lax.fori_loop(0, per_tile // 16, body, 0)
        pltpu.sync_copy(hist, shared.at[sid])
        plsc.subcore_barrier()
        for t in range(NS):
            pltpu.sync_copy(shared.at[t, pl.ds(sid * seg, seg)],
                            colbuf.at[t])

        def red(j, _):
            s = jnp.zeros((16,), jnp.float32)
            for t in range(NS):
                s = s + colbuf[t, pl.ds(j * 16, 16)]
            sumbuf[pl.ds(j * 16, 16)] = s
            return 0

        lax.fori_loop(0, seg // 16, red, 0)
        pltpu.sync_copy(sumbuf, out_hbm.at[pl.ds(sid * seg, seg)])

    @pl.when(cid == 0)
    def _():
        run(dst1, cnt1)

    @pl.when(cid == 1)
    def _():
        run(dst2, cnt2)


@functools.partial(
    pl.kernel,
    out_type=(
        jax.ShapeDtypeStruct((NP, DF), jnp.float32),
        jax.ShapeDtypeStruct((NP, DF), jnp.float32),
    ),
    mesh=_mesh,
    scratch_types=dict(
        acc_sh=pltpu.VMEM_SHARED((NP, DF), jnp.float32),
        sdb0=pltpu.VMEM((2, CH), jnp.int32),
        sdb1=pltpu.VMEM((2, CH), jnp.int32),
        sdb2=pltpu.VMEM((2, CH), jnp.int32),
        sdb3=pltpu.VMEM((2, CH), jnp.int32),
        rowsA=pltpu.VMEM((CH, DF), jnp.float32),
        rowsB=pltpu.VMEM((CH, DF), jnp.float32),
        gsA=pltpu.SemaphoreType.DMA,
        gsB=pltpu.SemaphoreType.DMA,
        ssA=pltpu.SemaphoreType.DMA,
        ssB=pltpu.SemaphoreType.DMA,
        is0=pltpu.SemaphoreType.DMA,
        is1=pltpu.SemaphoreType.DMA,
        is2=pltpu.SemaphoreType.DMA,
        is3=pltpu.SemaphoreType.DMA,
    ),
    compiler_params=pltpu.CompilerParams(needs_layout_passes=False),
)
def _agg_kernel(g1, g2, sd1, sd2, zrows, acc1, acc2,
                acc_sh, sdb0, sdb1, sdb2, sdb3, rowsA, rowsB,
                gsA, gsB, ssA, ssB, is0, is1, is2, is3):
    cid = lax.axis_index("c")
    sid = lax.axis_index("s")
    zr = NP // NS
    pltpu.sync_copy(zrows, acc_sh.at[pl.ds(sid * zr, zr)])
    plsc.subcore_barrier()

    sd = (sdb0, sdb1, sdb2, sdb3)
    rows = (rowsA, rowsB)
    gsem = (gsA, gsB)
    ssem = (ssA, ssB)
    isem = (is0, is1, is2, is3)

    def run(g_hbm, sd_hbm, out_hbm):
        pltpu.sync_copy(sd_hbm.at[sid], sdb0)
        pltpu.async_copy(g_hbm.at[sdb0.at[0]], rowsA, gsA)
        pltpu.async_copy(sd_hbm.at[NS + sid], sdb1, is1)
        pltpu.async_copy(sd_hbm.at[2 * NS + sid], sdb2, is2)

        def body(j, _):
            for m in range(4):
                c = (4 * j + m) * NS + sid
                p, q = m % 2, (m + 1) % 2
                sk, sk1, sk3 = sd[m], sd[(m + 1) % 4], sd[(m + 3) % 4]

                @pl.when(c < NCHUNK)
                def _():
                    pltpu.make_async_copy(
                        g_hbm.at[sk.at[0]], rows[p], gsem[p]).wait()
                    pltpu.async_copy(
                        rows[p], acc_sh.at[sk.at[1]], ssem[p], add=True)

                @pl.when(jnp.logical_and(c >= NS, c - NS < NCHUNK))
                def _():
                    pltpu.make_async_copy(
                        rows[q], acc_sh.at[sk3.at[1]], ssem[q]).wait()

                @pl.when(c + 3 * NS < NCHUNK)
                def _():
                    pltpu.async_copy(sd_hbm.at[c + 3 * NS], sk3,
                                     isem[(m + 3) % 4])

                @pl.when(c + NS < NCHUNK)
                def _():
                    pltpu.make_async_copy(
                        sd_hbm.at[c + NS], sk1, isem[(m + 1) % 4]).wait()
                    pltpu.async_copy(g_hbm.at[sk1.at[0]], rows[q], gsem[q])

            return 0

        lax.fori_loop(0, (FULL + 2 + 3) // 4, body, 0)
        plsc.subcore_barrier()
        orows = NP // NS
        pltpu.sync_copy(
            acc_sh.at[pl.ds(sid * orows, orows)],
            out_hbm.at[pl.ds(sid * orows, orows)],
        )

    @pl.when(cid == 0)
    def _():
        run(g1, sd1, acc1)

    @pl.when(cid == 1)
    def _():
        run(g2, sd2, acc2)


def _hg_body(x1, w1, c1, x2, w2, c2, h1, g1, h2, g2):
    for x, w, c, h, g in ((x1, w1, c1, h1, g1), (x2, w2, c2, h2, g2)):
        hv = jnp.dot(x[...], w[...], preferred_element_type=jnp.float32)
        dis = lax.rsqrt(c[...] + 1.0)
        h[...] = hv
        g[...] = hv * dis


def _hg(x1, W1, cnt1, x2, W2, cnt2):
    nb = 10
    rb = N // nb
    spec_x = pl.BlockSpec((rb, DF), lambda i: (i, 0))
    spec_w = pl.BlockSpec((DF, DF), lambda i: (0, 0))
    spec_c = pl.BlockSpec((rb, 1), lambda i: (i, 0))
    return pl.pallas_call(
        _hg_body,
        grid=(nb,),
        in_specs=[spec_x, spec_w, spec_c, spec_x, spec_w, spec_c],
        out_specs=[spec_x, spec_x, spec_x, spec_x],
        out_shape=[jax.ShapeDtypeStruct((N, DF), jnp.float32)] * 4,
    )(x1, W1, cnt1, x2, W2, cnt2)


def _mas_body(d_ref, w_ref, b_ref, len_ref, o_ref):
    y = jnp.dot(d_ref[0, 0], w_ref[0], preferred_element_type=jnp.float32)
    y = _leaky(y + b_ref[0])
    pos = lax.broadcasted_iota(jnp.int32, (L, 1), 0)
    y = jnp.where(pos < len_ref[0, 0], y, -1e30)
    o_ref[0, 0] = jnp.max(y, axis=0, keepdims=True)


def _mas(data, wts, bias, lens):
    return pl.pallas_call(
        _mas_body,
        grid=(4, B),
        in_specs=[
            pl.BlockSpec((1, 1, L, DD), lambda b, g: (b, g, 0, 0)),
            pl.BlockSpec((1, DD, DO), lambda b, g: (b, 0, 0)),
            pl.BlockSpec((1, 1, DO), lambda b, g: (b, 0, 0)),
            pl.BlockSpec((1, 1, L, 1), lambda b, g: (b, g, 0, 0)),
        ],
        out_specs=pl.BlockSpec((1, 1, 1, DO), lambda b, g: (b, g, 0, 0)),
        out_shape=jax.ShapeDtypeStruct((4, B, 1, DO), jnp.float32),
    )(data, wts, bias, lens)


def _combine_body(acc1, h1, c1, bt1, bg1, wf1, bf1,
                  acc2, h2, c2, bt2, bg2, wf2, bf2,
                  mas, wfin, bfin, out):
    feats = []
    for acc, h, c, bt, bg, wf, bf in (
        (acc1, h1, c1, bt1, bg1, wf1, bf1),
        (acc2, h2, c2, bt2, bg2, wf2, bf2),
    ):
        deg = c[...] + 1.0
        dis = lax.rsqrt(deg)
        xn = _leaky(dis * acc[...] + h[...] / deg + bg[...])
        gid = lax.broadcasted_iota(jnp.int32, (B, N), 0)
        m = (gid == bt[...]).astype(jnp.float32)
        sums = jnp.dot(m, xn, preferred_element_type=jnp.float32)
        cnts = jnp.sum(m, axis=1, keepdims=True)
        mean = sums / jnp.maximum(cnts, 1.0)
        feats.append(_leaky(jnp.dot(mean, wf[...],
                                    preferred_element_type=jnp.float32)
                            + bf[...]))
    for i in range(4):
        feats.append(mas[i, :, 0, :])
    comb = jnp.concatenate(feats, axis=1)
    out[...] = jnp.dot(comb, wfin[...],
                       preferred_element_type=jnp.float32) + bfin[...]


def _combine(acc1, h1, cnt1, bt1, bg1, wf1, bf1,
             acc2, h2, cnt2, bt2, bg2, wf2, bf2,
             mas, wfin, bfin):
    args = (acc1, h1, cnt1, bt1, bg1, wf1, bf1,
            acc2, h2, cnt2, bt2, bg2, wf2, bf2,
            mas, wfin, bfin)
    return pl.pallas_call(
        _combine_body,
        out_shape=jax.ShapeDtypeStruct((B, 1), jnp.float32),
    )(*args)


def kernel(pro1_x, pro1_edge_index, pro1_batch, pro2_x, pro2_edge_index,
           pro2_batch, mas1_straight, mas1_flipped, mas2_straight,
           mas2_flipped, mas1_straight_lengths, mas1_flipped_lengths,
           mas2_straight_lengths, mas2_flipped_lengths,
           W_gcn1, b_gcn1, W_gcn2, b_gcn2, W_fc1, b_fc1, W_fc2, b_fc2,
           W_m1s, b_m1s, W_m1f, b_m1f, W_m2s, b_m2s, W_m2f, b_m2f,
           W_final, b_final):
    src1, dst1 = pro1_edge_index[0], pro1_edge_index[1]
    src2, dst2 = pro2_edge_index[0], pro2_edge_index[1]

    zeros1d = jnp.zeros((NP,), jnp.float32)
    cnt1_p, cnt2_p = _deg_kernel(dst1, dst2, zeros1d)
    cnt1 = cnt1_p[:N].reshape(N, 1)
    cnt2 = cnt2_p[:N].reshape(N, 1)

    h1, g1, h2, g2 = _hg(pro1_x, W_gcn1, cnt1, pro2_x, W_gcn2, cnt2)

    sd1 = jnp.stack([src1.reshape(NCHUNK, CH), dst1.reshape(NCHUNK, CH)], 1)
    sd2 = jnp.stack([src2.reshape(NCHUNK, CH), dst2.reshape(NCHUNK, CH)], 1)

    zrows = jnp.zeros((NP // NS, DF), jnp.float32)
    acc1, acc2 = _agg_kernel(g1, g2, sd1, sd2, zrows)
    acc1, acc2 = acc1[:N], acc2[:N]

    mas_data = jnp.stack([mas1_straight, mas1_flipped,
                          mas2_straight, mas2_flipped])
    mas_w = jnp.stack([W_m1s.T, W_m1f.T, W_m2s.T, W_m2f.T])
    mas_b = jnp.stack([b_m1s, b_m1f, b_m2s, b_m2f]).reshape(4, 1, DO)
    mas_len = jnp.stack([mas1_straight_lengths, mas1_flipped_lengths,
                         mas2_straight_lengths, mas2_flipped_lengths])
    mas_len = jnp.broadcast_to(mas_len[:, :, None, None], (4, B, L, 1))
    mas_out = _mas(mas_data, mas_w, mas_b, mas_len)

    return _combine(
        acc1, h1, cnt1, pro1_batch.reshape(1, N), b_gcn1.reshape(1, DF),
        W_fc1.T, b_fc1.reshape(1, DO),
        acc2, h2, cnt2, pro2_batch.reshape(1, N), b_gcn2.reshape(1, DF),
        W_fc2.T, b_fc2.reshape(1, DO),
        mas_out, W_final.T, b_final.reshape(1, 1))


# speedup vs baseline: 1.4534x; 1.0007x over previous
import functools
import jax
import jax.numpy as jnp
from jax import lax
from jax.experimental import pallas as pl
from jax.experimental.pallas import tpu as pltpu
from jax.experimental.pallas import tpu_sc as plsc

N = 10000
E = 320000
B = 16
L = 256
DF = 128
DO = 128
DD = 80

NP = 10240
HR = 80
NC = 2
NS = 16
CH = 128
NCHUNK = E // CH
FULL = NCHUNK // NS
REM = NCHUNK - FULL * NS

_mesh = plsc.VectorSubcoreMesh(core_axis_name="c", subcore_axis_name="s")


def _leaky(x):
    return jnp.where(x >= 0, x, 0.01 * x)


@functools.partial(
    pl.kernel,
    out_type=(
        jax.ShapeDtypeStruct((NP,), jnp.float32),
        jax.ShapeDtypeStruct((NP,), jnp.float32),
    ),
    mesh=_mesh,
    scratch_types=dict(
        shared=pltpu.VMEM_SHARED((NS, NP), jnp.float32),
        hist=pltpu.VMEM((NP,), jnp.float32),
        idxbuf=pltpu.VMEM((E // NS,), jnp.int32),
        colbuf=pltpu.VMEM((NS, NP // NS), jnp.float32),
        sumbuf=pltpu.VMEM((NP // NS,), jnp.float32),
    ),
    compiler_params=pltpu.CompilerParams(needs_layout_passes=False),
)
def _deg_kernel(dst1, dst2, zeros1d, cnt1, cnt2,
                shared, hist, idxbuf, colbuf, sumbuf):
    cid = lax.axis_index("c")
    sid = lax.axis_index("s")
    per_tile = E // NS
    seg = NP // NS

    pltpu.sync_copy(zeros1d, hist)

    def run(dst_hbm, out_hbm):
        pltpu.sync_copy(dst_hbm.at[pl.ds(sid * per_tile, per_tile)], idxbuf)
        ones = jnp.full((16,), 1.0, jnp.float32)

        def body(i, _):
            idx = idxbuf[pl.ds(i * 16, 16)]
            plsc.addupdate_scatter(hist, [idx], ones)
            return 0

        lax.fori_loop(0, per_tile // 16, body, 0)
        pltpu.sync_copy(hist, shared.at[sid])
        plsc.subcore_barrier()
        for t in range(NS):
            pltpu.sync_copy(shared.at[t, pl.ds(sid * seg, seg)],
                            colbuf.at[t])

        def red(j, _):
            s = jnp.zeros((16,), jnp.float32)
            for t in range(NS):
                s = s + colbuf[t, pl.ds(j * 16, 16)]
            sumbuf[pl.ds(j * 16, 16)] = s
            return 0

        lax.fori_loop(0, seg // 16, red, 0)
        pltpu.sync_copy(sumbuf, out_hbm.at[pl.ds(sid * seg, seg)])

    @pl.when(cid == 0)
    def _():
        run(dst1, cnt1)

    @pl.when(cid == 1)
    def _():
        run(dst2, cnt2)


@functools.partial(
    pl.kernel,
    out_type=(
        jax.ShapeDtypeStruct((NP, DF), jnp.float32),
        jax.ShapeDtypeStruct((NP, DF), jnp.float32),
    ),
    mesh=_mesh,
    scratch_types=dict(
        acc_sh=pltpu.VMEM_SHARED((NP, DF), jnp.float32),
        sdb0=pltpu.VMEM((2, CH), jnp.int32),
        sdb1=pltpu.VMEM((2, CH), jnp.int32),
        sdb2=pltpu.VMEM((2, CH), jnp.int32),
        sdb3=pltpu.VMEM((2, CH), jnp.int32),
        rowsA=pltpu.VMEM((CH, DF), jnp.float32),
        rowsB=pltpu.VMEM((CH, DF), jnp.float32),
        gsA=pltpu.SemaphoreType.DMA,
        gsB=pltpu.SemaphoreType.DMA,
        ssA=pltpu.SemaphoreType.DMA,
        ssB=pltpu.SemaphoreType.DMA,
        is0=pltpu.SemaphoreType.DMA,
        is1=pltpu.SemaphoreType.DMA,
        is2=pltpu.SemaphoreType.DMA,
        is3=pltpu.SemaphoreType.DMA,
    ),
    compiler_params=pltpu.CompilerParams(needs_layout_passes=False),
)
def _agg_kernel(g1, g2, sd1, sd2, zrows, acc1, acc2,
                acc_sh, sdb0, sdb1, sdb2, sdb3, rowsA, rowsB,
                gsA, gsB, ssA, ssB, is0, is1, is2, is3):
    cid = lax.axis_index("c")
    sid = lax.axis_index("s")
    zr = NP // NS
    pltpu.sync_copy(zrows, acc_sh.at[pl.ds(sid * zr, zr)])
    plsc.subcore_barrier()

    sd = (sdb0, sdb1, sdb2, sdb3)
    rows = (rowsA, rowsB)
    gsem = (gsA, gsB)
    ssem = (ssA, ssB)
    isem = (is0, is1, is2, is3)

    def run(g_hbm, sd_hbm, out_hbm):
        pltpu.sync_copy(sd_hbm.at[sid], sdb0)
        pltpu.async_copy(g_hbm.at[sdb0.at[0]], rowsA, gsA)
        pltpu.async_copy(sd_hbm.at[NS + sid], sdb1, is1)
        pltpu.async_copy(sd_hbm.at[2 * NS + sid], sdb2, is2)

        def body(j, _):
            for m in range(4):
                c = (4 * j + m) * NS + sid
                p, q = m % 2, (m + 1) % 2
                sk, sk1, sk3 = sd[m], sd[(m + 1) % 4], sd[(m + 3) % 4]

                @pl.when(c < NCHUNK)
                def _():
                    pltpu.make_async_copy(
                        g_hbm.at[sk.at[0]], rows[p], gsem[p]).wait()

                @pl.when(jnp.logical_and(c >= NS, c - NS < NCHUNK))
                def _():
                    pltpu.make_async_copy(
                        rows[q], acc_sh.at[sk3.at[1]], ssem[q]).wait()

                @pl.when(c + NS < NCHUNK)
                def _():
                    pltpu.make_async_copy(
                        sd_hbm.at[c + NS], sk1, isem[(m + 1) % 4]).wait()
                    pltpu.async_copy(g_hbm.at[sk1.at[0]], rows[q], gsem[q])

                @pl.when(c < NCHUNK)
                def _():
                    pltpu.async_copy(
                        rows[p], acc_sh.at[sk.at[1]], ssem[p], add=True)

                @pl.when(c + 3 * NS < NCHUNK)
                def _():
                    pltpu.async_copy(sd_hbm.at[c + 3 * NS], sk3,
                                     isem[(m + 3) % 4])

            return 0

        lax.fori_loop(0, (FULL + 2 + 3) // 4, body, 0)
        plsc.subcore_barrier()
        orows = NP // NS
        pltpu.sync_copy(
            acc_sh.at[pl.ds(sid * orows, orows)],
            out_hbm.at[pl.ds(sid * orows, orows)],
        )

    @pl.when(cid == 0)
    def _():
        run(g1, sd1, acc1)

    @pl.when(cid == 1)
    def _():
        run(g2, sd2, acc2)


def _hg_body(x1, w1, c1, x2, w2, c2, h1, g1, h2, g2):
    for x, w, c, h, g in ((x1, w1, c1, h1, g1), (x2, w2, c2, h2, g2)):
        hv = jnp.dot(x[...], w[...], preferred_element_type=jnp.float32)
        dis = lax.rsqrt(c[...] + 1.0)
        h[...] = hv
        g[...] = hv * dis


def _hg(x1, W1, cnt1, x2, W2, cnt2):
    nb = 10
    rb = N // nb
    spec_x = pl.BlockSpec((rb, DF), lambda i: (i, 0))
    spec_w = pl.BlockSpec((DF, DF), lambda i: (0, 0))
    spec_c = pl.BlockSpec((rb, 1), lambda i: (i, 0))
    return pl.pallas_call(
        _hg_body,
        grid=(nb,),
        in_specs=[spec_x, spec_w, spec_c, spec_x, spec_w, spec_c],
        out_specs=[spec_x, spec_x, spec_x, spec_x],
        out_shape=[jax.ShapeDtypeStruct((N, DF), jnp.float32)] * 4,
    )(x1, W1, cnt1, x2, W2, cnt2)


def _mas_body(d_ref, w_ref, b_ref, len_ref, o_ref):
    y = jnp.dot(d_ref[0, 0], w_ref[0], preferred_element_type=jnp.float32)
    y = _leaky(y + b_ref[0])
    pos = lax.broadcasted_iota(jnp.int32, (L, 1), 0)
    y = jnp.where(pos < len_ref[0, 0], y, -1e30)
    o_ref[0, 0] = jnp.max(y, axis=0, keepdims=True)


def _mas(data, wts, bias, lens):
    return pl.pallas_call(
        _mas_body,
        grid=(4, B),
        in_specs=[
            pl.BlockSpec((1, 1, L, DD), lambda b, g: (b, g, 0, 0)),
            pl.BlockSpec((1, DD, DO), lambda b, g: (b, 0, 0)),
            pl.BlockSpec((1, 1, DO), lambda b, g: (b, 0, 0)),
            pl.BlockSpec((1, 1, L, 1), lambda b, g: (b, g, 0, 0)),
        ],
        out_specs=pl.BlockSpec((1, 1, 1, DO), lambda b, g: (b, g, 0, 0)),
        out_shape=jax.ShapeDtypeStruct((4, B, 1, DO), jnp.float32),
    )(data, wts, bias, lens)


def _combine_body(acc1, h1, c1, bt1, bg1, wf1, bf1,
                  acc2, h2, c2, bt2, bg2, wf2, bf2,
                  mas, wfin, bfin, out):
    feats = []
    for acc, h, c, bt, bg, wf, bf in (
        (acc1, h1, c1, bt1, bg1, wf1, bf1),
        (acc2, h2, c2, bt2, bg2, wf2, bf2),
    ):
        deg = c[...] + 1.0
        dis = lax.rsqrt(deg)
        xn = _leaky(dis * acc[...] + h[...] / deg + bg[...])
        gid = lax.broadcasted_iota(jnp.int32, (B, N), 0)
        m = (gid == bt[...]).astype(jnp.float32)
        sums = jnp.dot(m, xn, preferred_element_type=jnp.float32)
        cnts = jnp.sum(m, axis=1, keepdims=True)
        mean = sums / jnp.maximum(cnts, 1.0)
        feats.append(_leaky(jnp.dot(mean, wf[...],
                                    preferred_element_type=jnp.float32)
                            + bf[...]))
    for i in range(4):
        feats.append(mas[i, :, 0, :])
    comb = jnp.concatenate(feats, axis=1)
    out[...] = jnp.dot(comb, wfin[...],
                       preferred_element_type=jnp.float32) + bfin[...]


def _combine(acc1, h1, cnt1, bt1, bg1, wf1, bf1,
             acc2, h2, cnt2, bt2, bg2, wf2, bf2,
             mas, wfin, bfin):
    args = (acc1, h1, cnt1, bt1, bg1, wf1, bf1,
            acc2, h2, cnt2, bt2, bg2, wf2, bf2,
            mas, wfin, bfin)
    return pl.pallas_call(
        _combine_body,
        out_shape=jax.ShapeDtypeStruct((B, 1), jnp.float32),
    )(*args)


def kernel(pro1_x, pro1_edge_index, pro1_batch, pro2_x, pro2_edge_index,
           pro2_batch, mas1_straight, mas1_flipped, mas2_straight,
           mas2_flipped, mas1_straight_lengths, mas1_flipped_lengths,
           mas2_straight_lengths, mas2_flipped_lengths,
           W_gcn1, b_gcn1, W_gcn2, b_gcn2, W_fc1, b_fc1, W_fc2, b_fc2,
           W_m1s, b_m1s, W_m1f, b_m1f, W_m2s, b_m2s, W_m2f, b_m2f,
           W_final, b_final):
    src1, dst1 = pro1_edge_index[0], pro1_edge_index[1]
    src2, dst2 = pro2_edge_index[0], pro2_edge_index[1]

    zeros1d = jnp.zeros((NP,), jnp.float32)
    cnt1_p, cnt2_p = _deg_kernel(dst1, dst2, zeros1d)
    cnt1 = cnt1_p[:N].reshape(N, 1)
    cnt2 = cnt2_p[:N].reshape(N, 1)

    h1, g1, h2, g2 = _hg(pro1_x, W_gcn1, cnt1, pro2_x, W_gcn2, cnt2)

    sd1 = jnp.stack([src1.reshape(NCHUNK, CH), dst1.reshape(NCHUNK, CH)], 1)
    sd2 = jnp.stack([src2.reshape(NCHUNK, CH), dst2.reshape(NCHUNK, CH)], 1)

    zrows = jnp.zeros((NP // NS, DF), jnp.float32)
    acc1, acc2 = _agg_kernel(g1, g2, sd1, sd2, zrows)
    acc1, acc2 = acc1[:N], acc2[:N]

    mas_data = jnp.stack([mas1_straight, mas1_flipped,
                          mas2_straight, mas2_flipped])
    mas_w = jnp.stack([W_m1s.T, W_m1f.T, W_m2s.T, W_m2f.T])
    mas_b = jnp.stack([b_m1s, b_m1f, b_m2s, b_m2f]).reshape(4, 1, DO)
    mas_len = jnp.stack([mas1_straight_lengths, mas1_flipped_lengths,
                         mas2_straight_lengths, mas2_flipped_lengths])
    mas_len = jnp.broadcast_to(mas_len[:, :, None, None], (4, B, L, 1))
    mas_out = _mas(mas_data, mas_w, mas_b, mas_len)

    return _combine(
        acc1, h1, cnt1, pro1_batch.reshape(1, N), b_gcn1.reshape(1, DF),
        W_fc1.T, b_fc1.reshape(1, DO),
        acc2, h2, cnt2, pro2_batch.reshape(1, N), b_gcn2.reshape(1, DF),
        W_fc2.T, b_fc2.reshape(1, DO),
        mas_out, W_final.T, b_final.reshape(1, 1))
